# baseline (speedup 1.0000x reference)
import functools

import jax
import jax.numpy as jnp
from jax import lax
from jax.experimental import pallas as pl
from jax.experimental.pallas import tpu as pltpu
from jax.experimental.pallas import tpu_sc as plsc

NY, NX, NE = 256, 256, 128
NS = 2048
BATCH = 4
FS = 40000000.0
OFFSET = 10.0

P = NY * NX
L = 16
NCHUNK = NE // L
NP = 1024
NBLK = P // NP // 4
BR = NP // NX


def _make_sc_kernel(b):

  def _sc_kernel(sin_hbm, scale_hbm, dist_hbm, out_hbm,
                   table, distbuf, outbuf, scalebuf, dsem, osem):
      c = lax.axis_index("c")
      s = lax.axis_index("s")
      wid = s * 2 + c
      q = wid // NCHUNK
      e0 = (wid % NCHUNK) * L

      pltpu.sync_copy(scale_hbm.at[b], scalebuf)
      pltpu.sync_copy(sin_hbm.at[b, :, pl.ds(e0, L)], table)

      lane = lax.iota(jnp.int32, L)
      scale = scalebuf[...]

      def dist_slice(blk):
          return dist_hbm.at[pl.ds((q * NBLK + blk) * NP, NP), pl.ds(e0, L)]

      def out_slice(blk):
          return out_hbm.at[0, pl.ds(e0, L),
                            pl.ds((q * NBLK + blk) * BR, BR), :]

      def out_tile(buf):
          return outbuf.at[buf, :, pl.ds(0, BR), pl.ds(0, NX)]

      def start_dist(blk, buf):
          pltpu.async_copy(dist_slice(blk), distbuf.at[buf], dsem.at[buf])

      def compute(buf):
          dbuf = distbuf.at[buf]
          obuf = outbuf.at[buf]

          for r in range(BR):
              row = jnp.full((L,), r, jnp.int32)

              @plsc.parallel_loop(0, NX, unroll=8)
              def _pixels(i):
                  d = dbuf[r * NX + i]
                  tof = d * scale - OFFSET
                  f = tof.astype(jnp.int32)
                  w = tof - f.astype(jnp.float32)
                  gf = plsc.load_gather(table, [f, lane])
                  gc = plsc.load_gather(table, [f + 1, lane])
                  res = gf + w * (gc - gf)
                  plsc.store_scatter(
                      obuf, [lane, row, jnp.full((L,), i, jnp.int32)], res)

      start_dist(0, 0)
      start_dist(1, 1)

      @pl.loop(0, NBLK, step=2)
      def _blocks(blk):
          for buf in (0, 1):
              blkb = blk + buf
              pltpu.make_async_copy(dist_slice(blkb), distbuf.at[buf],
                                    dsem.at[buf]).wait()

              @pl.when(blkb >= 2)
              def _():
                  pltpu.make_async_copy(out_tile(buf), out_slice(blkb - 2),
                                        osem.at[buf]).wait()

              compute(buf)

              @pl.when(blkb + 2 < NBLK)
              def _():
                  start_dist(blkb + 2, buf)

              pltpu.async_copy(out_tile(buf), out_slice(blkb), osem.at[buf])

      pltpu.make_async_copy(out_tile(0), out_slice(NBLK - 2), osem.at[0]).wait()
      pltpu.make_async_copy(out_tile(1), out_slice(NBLK - 1), osem.at[1]).wait()

  return _sc_kernel


def kernel(input_sinogram, sos, transducer_pixel_distances):
    scale = jnp.broadcast_to((FS / sos)[:, None], (BATCH, L))
    dist2d = transducer_pixel_distances.reshape(P, NE)

    mesh = plsc.VectorSubcoreMesh(core_axis_name="c", subcore_axis_name="s")
    slabs = []
    for b in range(BATCH):
        run = pl.kernel(
            _make_sc_kernel(b),
            out_type=jax.ShapeDtypeStruct((1, NE, NY, NX), jnp.float32),
            mesh=mesh,
            compiler_params=pltpu.CompilerParams(
                use_tc_tiling_on_sc=False, needs_layout_passes=False),
            scratch_types=[
                pltpu.VMEM((NS, L), jnp.float32),
                pltpu.VMEM((2, NP, L), jnp.float32),
                pltpu.VMEM((2, L, BR + 1, NX + 1), jnp.float32),
                pltpu.VMEM((L,), jnp.float32),
                pltpu.SemaphoreType.DMA((2,)),
                pltpu.SemaphoreType.DMA((2,)),
            ],
        )
        slabs.append(run(input_sinogram, scale, dist2d))
    out = jnp.zeros((BATCH, NE, NY, NX), jnp.float32)
    for b, slab in enumerate(slabs):
        out = lax.dynamic_update_slice(out, slab, (b, 0, 0, 0))
    return out


# speedup vs baseline: 1.1757x; 1.1757x over previous
import functools

import jax
import jax.numpy as jnp
from jax import lax
from jax.experimental import pallas as pl
from jax.experimental.pallas import tpu as pltpu
from jax.experimental.pallas import tpu_sc as plsc

NY, NX, NE = 256, 256, 128
NS = 2048
BATCH = 4
FS = 40000000.0
OFFSET = 10.0

P = NY * NX
L = 16
NCHUNK = NE // L
NP = 1024
NBLK = P // NP
BR = NP // NX


def _sc_kernel(sin_hbm, scale_hbm, dist_hbm, out_hbm,
               table, distbuf, outbuf, scalebuf, dsem, osem):
    c = lax.axis_index("c")
    s = lax.axis_index("s")
    wid = s * 2 + c
    b = wid // NCHUNK
    e0 = (wid % NCHUNK) * L

    pltpu.sync_copy(scale_hbm.at[b], scalebuf)
    pltpu.sync_copy(sin_hbm.at[b, :, pl.ds(e0, L)], table)

    lane = lax.iota(jnp.int32, L)
    scale = scalebuf[...]

    def dist_slice(blk):
        return dist_hbm.at[pl.ds(blk * NP, NP), pl.ds(e0, L)]

    def out_slice(blk):
        return out_hbm.at[b, pl.ds(e0, L), pl.ds(blk * BR, BR), :]

    def out_tile(buf):
        return outbuf.at[buf, :, pl.ds(0, BR), pl.ds(0, NX)]

    def start_dist(blk, buf):
        pltpu.async_copy(dist_slice(blk), distbuf.at[buf], dsem.at[buf])

    def compute(buf):
        dbuf = distbuf.at[buf]
        obuf = outbuf.at[buf]

        for r in range(BR):
            row = jnp.full((L,), r, jnp.int32)

            @plsc.parallel_loop(0, NX, unroll=8)
            def _pixels(i):
                d = dbuf[r * NX + i]
                tof = d * scale - OFFSET
                f = tof.astype(jnp.int32)
                w = tof - f.astype(jnp.float32)
                gf = plsc.load_gather(table, [f, lane])
                gc = plsc.load_gather(table, [f + 1, lane])
                res = gf + w * (gc - gf)
                plsc.store_scatter(
                    obuf, [lane, row, jnp.full((L,), i, jnp.int32)], res)

    start_dist(0, 0)
    start_dist(1, 1)

    @pl.loop(0, NBLK, step=2)
    def _blocks(blk):
        for buf in (0, 1):
            blkb = blk + buf
            pltpu.make_async_copy(dist_slice(blkb), distbuf.at[buf],
                                  dsem.at[buf]).wait()

            @pl.when(blkb >= 2)
            def _():
                pltpu.make_async_copy(out_tile(buf), out_slice(blkb - 2),
                                      osem.at[buf]).wait()

            compute(buf)

            @pl.when(blkb + 2 < NBLK)
            def _():
                start_dist(blkb + 2, buf)

            pltpu.async_copy(out_tile(buf), out_slice(blkb), osem.at[buf])

    pltpu.make_async_copy(out_tile(0), out_slice(NBLK - 2), osem.at[0]).wait()
    pltpu.make_async_copy(out_tile(1), out_slice(NBLK - 1), osem.at[1]).wait()


def kernel(input_sinogram, sos, transducer_pixel_distances):
    scale = jnp.broadcast_to((FS / sos)[:, None], (BATCH, L))
    dist2d = transducer_pixel_distances.reshape(P, NE)

    mesh = plsc.VectorSubcoreMesh(core_axis_name="c", subcore_axis_name="s")
    run = pl.kernel(
        _sc_kernel,
        out_type=jax.ShapeDtypeStruct((BATCH, NE, NY, NX), jnp.float32),
        mesh=mesh,
        compiler_params=pltpu.CompilerParams(
            use_tc_tiling_on_sc=False, needs_layout_passes=False),
        scratch_types=[
            pltpu.VMEM((NS, L), jnp.float32),
            pltpu.VMEM((2, NP, L), jnp.float32),
            pltpu.VMEM((2, L, BR + 1, NX + 1), jnp.float32),
            pltpu.VMEM((L,), jnp.float32),
            pltpu.SemaphoreType.DMA((2,)),
            pltpu.SemaphoreType.DMA((2,)),
        ],
    )
    return run(input_sinogram, scale, dist2d)
